# Initial kernel scaffold; baseline (speedup 1.0000x reference)
import jax
import jax.numpy as jnp
from jax.experimental import pallas as pl


def kernel(x, fea, W1, b1, g1, be1, W2, b2, g2, be2, W3, b3):
    raise NotImplementedError("write your pallas kernel here")


# speedup vs baseline: 85.7752x; 85.7752x over previous
import functools

import jax
import jax.numpy as jnp
from jax.experimental import pallas as pl
from jax.experimental.pallas import tpu as pltpu

BINS = 16
LO = 1.0
HI = 80.0
WIDTH = (HI - LO) / BINS
B, N, C = 4, 2048, 3
FEAT = 128
HID = FEAT // 2
ROWS = 256


def _hist_body(xi_ref, xj_ref, counts_ref):
    sq = None
    for c in range(C):
        d = xi_ref[0, :, c:c + 1] - xj_ref[0, c:c + 1, :]
        sq = d * d if sq is None else sq + d * d
    dist = jnp.sqrt(jnp.maximum(sq, 1e-24))
    idx = jnp.clip(jnp.floor((dist - LO) / WIDTH).astype(jnp.int32), 0, BINS - 1)
    valid = (dist >= LO) & (dist <= HI)
    cols = []
    for k in range(BINS):
        hit = jnp.where((idx == k) & valid, 1.0, 0.0)
        cols.append(jnp.sum(hit, axis=1, keepdims=True))
    counts_ref[...] = jnp.concatenate(cols, axis=1)


def _histograms(x):
    xt = jnp.transpose(x, (0, 2, 1))
    grid = (B, N // ROWS)
    return pl.pallas_call(
        _hist_body,
        grid=grid,
        in_specs=[
            pl.BlockSpec((1, ROWS, C), lambda b, r: (b, r, 0)),
            pl.BlockSpec((1, C, N), lambda b, r: (b, 0, 0)),
        ],
        out_specs=pl.BlockSpec((ROWS, BINS), lambda b, r: (b * (N // ROWS) + r, 0)),
        out_shape=jax.ShapeDtypeStruct((B * N, BINS), jnp.float32),
    )(x, xt)


def _mlp_body(counts_ref, fea_ref, W1_ref, b1_ref, g1_ref, be1_ref,
              W2_ref, b2_ref, g2_ref, be2_ref, W3_ref, b3_ref, out_ref):
    counts = counts_ref[...]
    s = jnp.sum(counts, axis=1, keepdims=True)
    hist = counts / s

    def bn(z, g, be):
        m = jnp.mean(z, axis=1, keepdims=True)
        v = jnp.mean((z - m) * (z - m), axis=1, keepdims=True)
        return (z - m) / jnp.sqrt(v + 1e-5) * g + be

    z1 = jax.lax.dot_general(W1_ref[...], hist, (((1,), (1,)), ((), ())),
                             preferred_element_type=jnp.float32) + b1_ref[...]
    h1 = jax.nn.relu(bn(z1, g1_ref[...], be1_ref[...]))
    z2 = jax.lax.dot_general(W2_ref[...], h1, (((1,), (0,)), ((), ())),
                             preferred_element_type=jnp.float32) + b2_ref[...]
    h2 = jax.nn.relu(bn(z2, g2_ref[...], be2_ref[...]))
    z3 = jax.lax.dot_general(W3_ref[...], h2, (((1,), (0,)), ((), ())),
                             preferred_element_type=jnp.float32) + b3_ref[...]
    for b in range(B):
        out_ref[b] = fea_ref[b] + z3[:, b * N:(b + 1) * N]


def kernel(x, fea, W1, b1, g1, be1, W2, b2, g2, be2, W3, b3):
    counts = _histograms(x)
    out = pl.pallas_call(
        _mlp_body,
        out_shape=jax.ShapeDtypeStruct((B, FEAT, N), jnp.float32),
    )(counts, fea, W1, b1.reshape(HID, 1), g1.reshape(HID, 1),
      be1.reshape(HID, 1), W2, b2.reshape(HID, 1), g2.reshape(HID, 1),
      be2.reshape(HID, 1), W3, b3.reshape(FEAT, 1))
    return out
